# Initial kernel scaffold; baseline (speedup 1.0000x reference)
import jax
import jax.numpy as jnp
from jax.experimental import pallas as pl


def kernel(input_ids, embed_weight, proj_weight):
    raise NotImplementedError("write your pallas kernel here")


# speedup vs baseline: 11.0099x; 11.0099x over previous
import functools

import jax
import jax.numpy as jnp
from jax import lax
from jax.experimental import pallas as pl
from jax.experimental.pallas import tpu as pltpu
from jax.experimental.pallas import tpu_sc as plsc

VOCAB = 1000000
D_MODEL = 768
R = 128
B = 4096
L = 200
BL = B * L

NC = 2
NS = 16
NW = NC * NS
PER_W = BL // NW
CHUNK = 512
N_SUB = CHUNK // 128
STEPS = PER_W // CHUNK


def _gather_body(ids_hbm, table_hbm, out_hbm, idx_v, rows_v, sem):
    c = lax.axis_index("c")
    s = lax.axis_index("s")
    wid = s * NC + c
    base_row = wid * (PER_W // 128)

    def step(g, _):
        row = base_row + g * N_SUB
        pltpu.sync_copy(ids_hbm.at[pl.ds(row, N_SUB)], idx_v)
        cps = [
            pltpu.async_copy(
                table_hbm.at[idx_v.at[j]],
                rows_v.at[pl.ds(j * 128, 128)],
                sem,
            )
            for j in range(N_SUB)
        ]
        for cp in cps:
            cp.wait()
        pltpu.sync_copy(
            rows_v, out_hbm.at[pl.ds(wid * PER_W + g * CHUNK, CHUNK)]
        )

    pl.loop(0, STEPS)(lambda g: step(g, None))


@jax.jit
def _sc_gather(ids2d, table):
    kern = pl.kernel(
        _gather_body,
        out_type=jax.ShapeDtypeStruct((BL, R), jnp.float32),
        mesh=plsc.VectorSubcoreMesh(core_axis_name="c", subcore_axis_name="s"),
        scratch_types=[
            pltpu.VMEM((N_SUB, 128), jnp.int32),
            pltpu.VMEM((CHUNK, R), jnp.float32),
            pltpu.SemaphoreType.DMA,
        ],
    )
    return kern(ids2d, table)


RB = 1024


def _proj_body(x_ref, w_ref, o_ref):
    o_ref[...] = jnp.dot(
        x_ref[...], w_ref[...], preferred_element_type=jnp.float32
    )


@jax.jit
def _tc_project(x, w_t):
    return pl.pallas_call(
        _proj_body,
        grid=(BL // RB,),
        in_specs=[
            pl.BlockSpec((RB, R), lambda i: (i, 0)),
            pl.BlockSpec((R, D_MODEL), lambda i: (0, 0)),
        ],
        out_specs=pl.BlockSpec((RB, D_MODEL), lambda i: (i, 0)),
        out_shape=jax.ShapeDtypeStruct((BL, D_MODEL), jnp.float32),
        compiler_params=pltpu.CompilerParams(
            dimension_semantics=("arbitrary",),
        ),
    )(x, w_t)


def kernel(input_ids, embed_weight, proj_weight):
    ids2d = input_ids.reshape(BL // 128, 128).astype(jnp.int32)
    gathered = _sc_gather(ids2d, embed_weight)
    out = _tc_project(gathered, proj_weight.T)
    return out.reshape(B, L, D_MODEL)


# speedup vs baseline: 13.2879x; 1.2069x over previous
import functools

import jax
import jax.numpy as jnp
from jax import lax
from jax.experimental import pallas as pl
from jax.experimental.pallas import tpu as pltpu
from jax.experimental.pallas import tpu_sc as plsc

VOCAB = 1000000
D_MODEL = 768
R = 128
B = 4096
L = 200
BL = B * L

NC = 2
NS = 16
NW = NC * NS
PER_W = BL // NW
CHUNK = 256
N_SUB = CHUNK // 128
STEPS = PER_W // CHUNK


NBUF = 2


def _gather_body(ids_hbm, table_hbm, out_hbm, idx_v, rows_v, gsem, osem):
    c = lax.axis_index("c")
    s = lax.axis_index("s")
    wid = s * NC + c
    base_row = wid * (PER_W // 128)

    def fire(g, b):
        row = base_row + g * N_SUB
        pltpu.sync_copy(ids_hbm.at[pl.ds(row, N_SUB)], idx_v.at[b])
        for j in range(N_SUB):
            pltpu.async_copy(
                table_hbm.at[idx_v.at[b].at[j]],
                rows_v.at[b].at[pl.ds(j * 128, 128)],
                gsem,
            )

    def drain_and_store(g, b):
        for j in range(N_SUB):
            pltpu.make_async_copy(
                table_hbm.at[idx_v.at[b].at[j]],
                rows_v.at[b].at[pl.ds(j * 128, 128)],
                gsem,
            ).wait()
        pltpu.async_copy(
            rows_v.at[b],
            out_hbm.at[pl.ds(wid * PER_W + g * CHUNK, CHUNK)],
            osem,
        )

    def wait_store(g, b):
        pltpu.make_async_copy(
            rows_v.at[b],
            out_hbm.at[pl.ds(wid * PER_W + g * CHUNK, CHUNK)],
            osem,
        ).wait()

    fire(0, 0)

    def step(g):
        @pl.when(g >= 2)
        def _():
            wait_store(g - 1, 1)

        fire(g + 1, 1)
        drain_and_store(g, 0)

        @pl.when(g + 2 < STEPS)
        def _():
            wait_store(g, 0)
            fire(g + 2, 0)

        drain_and_store(g + 1, 1)

    pl.loop(0, STEPS, step=NBUF)(step)
    wait_store(STEPS - 2, 0)
    wait_store(STEPS - 1, 1)


@jax.jit
def _sc_gather(ids2d, table):
    kern = pl.kernel(
        _gather_body,
        out_type=jax.ShapeDtypeStruct((BL, R), jnp.float32),
        mesh=plsc.VectorSubcoreMesh(core_axis_name="c", subcore_axis_name="s"),
        scratch_types=[
            pltpu.VMEM((NBUF, N_SUB, 128), jnp.int32),
            pltpu.VMEM((NBUF, CHUNK, R), jnp.float32),
            pltpu.SemaphoreType.DMA,
            pltpu.SemaphoreType.DMA,
        ],
    )
    return kern(ids2d, table)


RB = 2048


def _proj_body(x_ref, w_ref, o_ref):
    o_ref[...] = jnp.dot(
        x_ref[...], w_ref[...], preferred_element_type=jnp.float32
    )


@jax.jit
def _tc_project(x, w_t):
    return pl.pallas_call(
        _proj_body,
        grid=(BL // RB,),
        in_specs=[
            pl.BlockSpec((RB, R), lambda i: (i, 0)),
            pl.BlockSpec((R, D_MODEL), lambda i: (0, 0)),
        ],
        out_specs=pl.BlockSpec((RB, D_MODEL), lambda i: (i, 0)),
        out_shape=jax.ShapeDtypeStruct((BL, D_MODEL), jnp.float32),
        compiler_params=pltpu.CompilerParams(
            dimension_semantics=("arbitrary",),
        ),
    )(x, w_t)


def kernel(input_ids, embed_weight, proj_weight):
    ids2d = input_ids.reshape(BL // 128, 128).astype(jnp.int32)
    gathered = _sc_gather(ids2d, embed_weight)
    out = _tc_project(gathered, proj_weight.T)
    return out.reshape(B, L, D_MODEL)
